# baseline (speedup 1.0000x reference)
import functools

import jax
import jax.numpy as jnp
from jax import lax
from jax.experimental import pallas as pl
from jax.experimental.pallas import tpu as pltpu
from jax.experimental.pallas import tpu_sc as plsc

B, S, H = 2, 2048, 1024
E, K = 16, 2
FE, FS = 512, 1408
T = B * S
N = T * K
M = 256
NT = N // M + E
PN = NT * M

NC, NS = 2, 16
NW = NC * NS


def _router_body(x_ref, wg_ref, oi_ref, ow_ref):
    logits = lax.dot_general(x_ref[...], wg_ref[...], (((1,), (0,)), ((), ())),
                             preferred_element_type=jnp.float32)
    lane = lax.broadcasted_iota(jnp.int32, logits.shape, 1)
    logits = jnp.where(lane < E, logits, -1e30)
    m1 = jnp.max(logits, axis=1, keepdims=True)
    i1 = jnp.min(jnp.where(logits == m1, lane, 127), axis=1, keepdims=True)
    l2 = jnp.where(lane == i1, -1e30, logits)
    m2 = jnp.max(l2, axis=1, keepdims=True)
    i2 = jnp.min(jnp.where(l2 == m2, lane, 127), axis=1, keepdims=True)
    e2 = jnp.exp(m2 - m1)
    wa = 1.0 / (1.0 + e2)
    wb = 1.0 - wa
    oi_ref[...] = jnp.where(lane == 0, i1, jnp.where(lane == 1, i2, 0))
    ow_ref[...] = jnp.where(lane == 0, wa, jnp.where(lane == 1, wb, 0.0))


def _router(xf, wgp):
    bt = 512
    return pl.pallas_call(
        _router_body,
        grid=(T // bt,),
        in_specs=[
            pl.BlockSpec((bt, H), lambda i: (i, 0)),
            pl.BlockSpec((H, 128), lambda i: (0, 0)),
        ],
        out_specs=[
            pl.BlockSpec((bt, 128), lambda i: (i, 0)),
            pl.BlockSpec((bt, 128), lambda i: (i, 0)),
        ],
        out_shape=[
            jax.ShapeDtypeStruct((T, 128), jnp.int32),
            jax.ShapeDtypeStruct((T, 128), jnp.float32),
        ],
    )(xf, wgp)


_SC_MESH = plsc.VectorSubcoreMesh(core_axis_name="c", subcore_axis_name="s",
                                  num_cores=NC, num_subcores=NS)
_D_CH = 64


@functools.partial(
    pl.kernel, mesh=_SC_MESH,
    out_type=jax.ShapeDtypeStruct((PN, H), jnp.float32),
    scratch_types=[
        pltpu.VMEM((_D_CH,), jnp.int32),
        pltpu.VMEM((_D_CH, H), jnp.float32),
        pltpu.SemaphoreType.DMA,
    ],
)
def _dispatch(xf_hbm, src_hbm, xg_hbm, idx_v, rows_v, sem):
    wid = lax.axis_index("s") * NC + lax.axis_index("c")
    per_w = PN // NW
    base = wid * per_w
    for c in range(per_w // _D_CH):
        off = base + c * _D_CH
        pltpu.sync_copy(src_hbm.at[pl.ds(off, _D_CH)], idx_v)
        pltpu.async_copy(xf_hbm.at[idx_v], rows_v, sem).wait()
        pltpu.sync_copy(rows_v, xg_hbm.at[pl.ds(off, _D_CH)])


def _ffn_body(te_ref, x_ref, w_ref, w1_ref, w3_ref, w2_ref, o_ref):
    del te_ref
    x = x_ref[...]
    h1 = lax.dot_general(x, w1_ref[0], (((1,), (1,)), ((), ())),
                         preferred_element_type=jnp.float32)
    h3 = lax.dot_general(x, w3_ref[0], (((1,), (1,)), ((), ())),
                         preferred_element_type=jnp.float32)
    w = w_ref[0, 0, :][:, None]
    hh = h1 * lax.logistic(h1) * h3 * w
    o_ref[...] = lax.dot_general(hh, w2_ref[0], (((1,), (1,)), ((), ())),
                                 preferred_element_type=jnp.float32)


def _ffn(tile_expert, xg, wslot3, w1, w3, w2):
    grid_spec = pltpu.PrefetchScalarGridSpec(
        num_scalar_prefetch=1,
        grid=(NT,),
        in_specs=[
            pl.BlockSpec((M, H), lambda i, te: (i, 0)),
            pl.BlockSpec((1, 1, M), lambda i, te: (i, 0, 0)),
            pl.BlockSpec((1, FE, H), lambda i, te: (te[i], 0, 0)),
            pl.BlockSpec((1, FE, H), lambda i, te: (te[i], 0, 0)),
            pl.BlockSpec((1, H, FE), lambda i, te: (te[i], 0, 0)),
        ],
        out_specs=pl.BlockSpec((M, H), lambda i, te: (i, 0)),
    )
    return pl.pallas_call(
        _ffn_body,
        grid_spec=grid_spec,
        out_shape=jax.ShapeDtypeStruct((PN, H), jnp.float32),
    )(tile_expert, xg, wslot3, w1, w3, w2)


def _shared_body(x_ref, w1_ref, w3_ref, w2_ref, o_ref):
    x = x_ref[...]
    h1 = lax.dot_general(x, w1_ref[...], (((1,), (1,)), ((), ())),
                         preferred_element_type=jnp.float32)
    h3 = lax.dot_general(x, w3_ref[...], (((1,), (1,)), ((), ())),
                         preferred_element_type=jnp.float32)
    hh = h1 * lax.logistic(h1) * h3
    o_ref[...] = lax.dot_general(hh, w2_ref[...], (((1,), (1,)), ((), ())),
                                 preferred_element_type=jnp.float32)


def _shared(xf, ws1, ws3, ws2):
    bt = 512
    return pl.pallas_call(
        _shared_body,
        grid=(T // bt,),
        in_specs=[
            pl.BlockSpec((bt, H), lambda i: (i, 0)),
            pl.BlockSpec((FS, H), lambda i: (0, 0)),
            pl.BlockSpec((FS, H), lambda i: (0, 0)),
            pl.BlockSpec((H, FS), lambda i: (0, 0)),
        ],
        out_specs=pl.BlockSpec((bt, H), lambda i: (i, 0)),
        out_shape=jax.ShapeDtypeStruct((T, H), jnp.float32),
    )(xf, ws1, ws3, ws2)


_C_CH = 32


@functools.partial(
    pl.kernel, mesh=_SC_MESH,
    out_type=jax.ShapeDtypeStruct((T, H), jnp.float32),
    scratch_types=[
        pltpu.VMEM((_C_CH,), jnp.int32),
        pltpu.VMEM((_C_CH,), jnp.int32),
        pltpu.VMEM((_C_CH, H), jnp.float32),
        pltpu.VMEM((_C_CH, H), jnp.float32),
        pltpu.VMEM((_C_CH, H), jnp.float32),
        pltpu.SemaphoreType.DMA,
    ],
)
def _combine(sh_hbm, yg_hbm, c0_hbm, c1_hbm, out_hbm,
             i0_v, i1_v, sh_v, y0_v, y1_v, sem):
    wid = lax.axis_index("s") * NC + lax.axis_index("c")
    per_w = T // NW
    base = wid * per_w
    for c in range(per_w // _C_CH):
        off = base + c * _C_CH
        pltpu.sync_copy(c0_hbm.at[pl.ds(off, _C_CH)], i0_v)
        pltpu.sync_copy(c1_hbm.at[pl.ds(off, _C_CH)], i1_v)
        pltpu.sync_copy(sh_hbm.at[pl.ds(off, _C_CH)], sh_v)
        pltpu.async_copy(yg_hbm.at[i0_v], y0_v, sem).wait()
        pltpu.async_copy(yg_hbm.at[i1_v], y1_v, sem).wait()
        for r in range(_C_CH):
            def _add(j, _, r=r):
                sl = pl.ds(j * 16, 16)
                sh_v[r, sl] = sh_v[r, sl] + y0_v[r, sl] + y1_v[r, sl]
                return 0
            lax.fori_loop(0, H // 16, _add, 0)
        pltpu.sync_copy(sh_v, out_hbm.at[pl.ds(off, _C_CH)])


def kernel(x, Wg, W1, W2, W3, Ws1, Ws2, Ws3):
    xf = x.reshape(T, H)
    wgp = jnp.zeros((H, 128), jnp.float32).at[:, :E].set(Wg.T)
    topi_p, topw_p = _router(xf, wgp)
    topi = topi_p[:, :K]
    topw = topw_p[:, :K]

    flat_e = topi.reshape(-1)
    oh = (flat_e[:, None] == jnp.arange(E)[None, :]).astype(jnp.int32)
    rank = jnp.take_along_axis(jnp.cumsum(oh, axis=0), flat_e[:, None], 1)[:, 0] - 1
    counts = oh.sum(axis=0)
    padded = ((counts + M - 1) // M) * M
    pstart = jnp.concatenate([jnp.zeros(1, padded.dtype), jnp.cumsum(padded)])[:E]
    dest = (pstart[flat_e] + rank).astype(jnp.int32)
    src_tok = jnp.zeros(PN, jnp.int32).at[dest].set(jnp.arange(N, dtype=jnp.int32) // K)
    wslot3 = jnp.zeros(PN, jnp.float32).at[dest].set(topw.reshape(-1)).reshape(NT, 1, M)
    tile_expert = (jnp.sum(jnp.arange(NT)[:, None] * M >= pstart[None, :], axis=1)
                   - 1).astype(jnp.int32)
    c0 = dest[0::K]
    c1 = dest[1::K]

    xg = _dispatch(xf, src_tok)
    shared = _shared(xf, Ws1, Ws3, Ws2)
    yg = _ffn(tile_expert, xg, wslot3, W1, W3, W2)
    out = _combine(shared, yg, c0, c1)
    return out.reshape(B, S, H)


# speedup vs baseline: 1.0465x; 1.0465x over previous
import functools

import jax
import jax.numpy as jnp
from jax import lax
from jax.experimental import pallas as pl
from jax.experimental.pallas import tpu as pltpu
from jax.experimental.pallas import tpu_sc as plsc

B, S, H = 2, 2048, 1024
E, K = 16, 2
FE, FS = 512, 1408
T = B * S
N = T * K
M = 256
NT = N // M + E
PN = NT * M

NC, NS = 2, 16
NW = NC * NS


def _router_body(x_ref, wg_ref, oi_ref, ow_ref):
    logits = lax.dot_general(x_ref[...], wg_ref[...], (((1,), (0,)), ((), ())),
                             preferred_element_type=jnp.float32)
    lane = lax.broadcasted_iota(jnp.int32, logits.shape, 1)
    logits = jnp.where(lane < E, logits, -1e30)
    m1 = jnp.max(logits, axis=1, keepdims=True)
    i1 = jnp.min(jnp.where(logits == m1, lane, 127), axis=1, keepdims=True)
    l2 = jnp.where(lane == i1, -1e30, logits)
    m2 = jnp.max(l2, axis=1, keepdims=True)
    i2 = jnp.min(jnp.where(l2 == m2, lane, 127), axis=1, keepdims=True)
    e2 = jnp.exp(m2 - m1)
    wa = 1.0 / (1.0 + e2)
    wb = 1.0 - wa
    oi_ref[...] = jnp.where(lane == 0, i1, jnp.where(lane == 1, i2, 0))
    ow_ref[...] = jnp.where(lane == 0, wa, jnp.where(lane == 1, wb, 0.0))


def _router(xf, wgp):
    bt = 512
    return pl.pallas_call(
        _router_body,
        grid=(T // bt,),
        in_specs=[
            pl.BlockSpec((bt, H), lambda i: (i, 0)),
            pl.BlockSpec((H, 128), lambda i: (0, 0)),
        ],
        out_specs=[
            pl.BlockSpec((bt, 128), lambda i: (i, 0)),
            pl.BlockSpec((bt, 128), lambda i: (i, 0)),
        ],
        out_shape=[
            jax.ShapeDtypeStruct((T, 128), jnp.int32),
            jax.ShapeDtypeStruct((T, 128), jnp.float32),
        ],
    )(xf, wgp)


_SC_MESH = plsc.VectorSubcoreMesh(core_axis_name="c", subcore_axis_name="s",
                                  num_cores=NC, num_subcores=NS)


def _make_gather(n_out, table_rows, ch):
    per_w = n_out // NW
    nch = per_w // ch
    assert per_w % ch == 0 and n_out % NW == 0

    @functools.partial(
        pl.kernel, mesh=_SC_MESH,
        out_type=jax.ShapeDtypeStruct((n_out, H), jnp.float32),
        scratch_types=[
            pltpu.VMEM((per_w,), jnp.int32),
            pltpu.VMEM((ch, H), jnp.float32),
            pltpu.VMEM((ch, H), jnp.float32),
            pltpu.SemaphoreType.DMA,
            pltpu.SemaphoreType.DMA,
            pltpu.SemaphoreType.DMA,
            pltpu.SemaphoreType.DMA,
        ],
    )
    def gather_kernel(table_hbm, idx_hbm, out_hbm, idx_v, rows0, rows1,
                      gsem0, gsem1, ssem0, ssem1):
        wid = lax.axis_index("s") * NC + lax.axis_index("c")
        base = wid * per_w
        pltpu.sync_copy(idx_hbm.at[pl.ds(base, per_w)], idx_v)
        bufs = (rows0, rows1)
        gsems = (gsem0, gsem1)
        ssems = (ssem0, ssem1)

        def gather(c, b):
            return pltpu.async_copy(
                table_hbm.at[idx_v.at[pl.ds(c * ch, ch)]], bufs[b], gsems[b])

        stores = [None, None]
        gs = [gather(0, 0), None]
        for c in range(nch):
            b = c & 1
            nb = 1 - b
            if c + 1 < nch:
                if stores[nb] is not None:
                    stores[nb].wait()
                gs[nb] = gather(c + 1, nb)
            gs[b].wait()
            stores[b] = pltpu.async_copy(
                bufs[b], out_hbm.at[pl.ds(base + c * ch, ch)], ssems[b])
        for st in stores:
            if st is not None:
                st.wait()

    return gather_kernel


_dispatch = _make_gather(PN, T, 48)
_combine_gather = _make_gather(N, PN, 32)


def _ffn_body(te_ref, x_ref, w_ref, w1_ref, w3_ref, w2_ref, o_ref):
    del te_ref
    x = x_ref[...].astype(jnp.bfloat16)
    h1 = lax.dot_general(x, w1_ref[0].astype(jnp.bfloat16),
                         (((1,), (1,)), ((), ())),
                         preferred_element_type=jnp.float32)
    h3 = lax.dot_general(x, w3_ref[0].astype(jnp.bfloat16),
                         (((1,), (1,)), ((), ())),
                         preferred_element_type=jnp.float32)
    w = w_ref[0, 0, :][:, None]
    hh = h1 * lax.logistic(h1) * h3 * w
    o_ref[...] = lax.dot_general(hh.astype(jnp.bfloat16),
                                 w2_ref[0].astype(jnp.bfloat16),
                                 (((1,), (1,)), ((), ())),
                                 preferred_element_type=jnp.float32)


def _ffn(tile_expert, xg, wslot3, w1, w3, w2):
    grid_spec = pltpu.PrefetchScalarGridSpec(
        num_scalar_prefetch=1,
        grid=(NT,),
        in_specs=[
            pl.BlockSpec((M, H), lambda i, te: (i, 0)),
            pl.BlockSpec((1, 1, M), lambda i, te: (i, 0, 0)),
            pl.BlockSpec((1, FE, H), lambda i, te: (te[i], 0, 0)),
            pl.BlockSpec((1, FE, H), lambda i, te: (te[i], 0, 0)),
            pl.BlockSpec((1, H, FE), lambda i, te: (te[i], 0, 0)),
        ],
        out_specs=pl.BlockSpec((M, H), lambda i, te: (i, 0)),
    )
    return pl.pallas_call(
        _ffn_body,
        grid_spec=grid_spec,
        out_shape=jax.ShapeDtypeStruct((PN, H), jnp.float32),
    )(tile_expert, xg, wslot3, w1, w3, w2)


def _shared_body(x_ref, w1_ref, w3_ref, w2_ref, o_ref):
    x = x_ref[...].astype(jnp.bfloat16)
    h1 = lax.dot_general(x, w1_ref[...].astype(jnp.bfloat16),
                         (((1,), (1,)), ((), ())),
                         preferred_element_type=jnp.float32)
    h3 = lax.dot_general(x, w3_ref[...].astype(jnp.bfloat16),
                         (((1,), (1,)), ((), ())),
                         preferred_element_type=jnp.float32)
    hh = h1 * lax.logistic(h1) * h3
    o_ref[...] = lax.dot_general(hh.astype(jnp.bfloat16),
                                 w2_ref[...].astype(jnp.bfloat16),
                                 (((1,), (1,)), ((), ())),
                                 preferred_element_type=jnp.float32)


def _shared(xf, ws1, ws3, ws2):
    bt = 512
    return pl.pallas_call(
        _shared_body,
        grid=(T // bt,),
        in_specs=[
            pl.BlockSpec((bt, H), lambda i: (i, 0)),
            pl.BlockSpec((FS, H), lambda i: (0, 0)),
            pl.BlockSpec((FS, H), lambda i: (0, 0)),
            pl.BlockSpec((H, FS), lambda i: (0, 0)),
        ],
        out_specs=pl.BlockSpec((bt, H), lambda i: (i, 0)),
        out_shape=jax.ShapeDtypeStruct((T, H), jnp.float32),
    )(xf, ws1, ws3, ws2)


def _add_body(s_ref, y0_ref, y1_ref, o_ref):
    o_ref[...] = s_ref[...] + y0_ref[0] + y1_ref[0]


def _final_add(shared, yt2):
    bt = 512
    return pl.pallas_call(
        _add_body,
        grid=(T // bt,),
        in_specs=[
            pl.BlockSpec((bt, H), lambda i: (i, 0)),
            pl.BlockSpec((1, bt, H), lambda i: (0, i, 0)),
            pl.BlockSpec((1, bt, H), lambda i: (1, i, 0)),
        ],
        out_specs=pl.BlockSpec((bt, H), lambda i: (i, 0)),
        out_shape=jax.ShapeDtypeStruct((T, H), jnp.float32),
    )(shared, yt2, yt2)


def kernel(x, Wg, W1, W2, W3, Ws1, Ws2, Ws3):
    xf = x.reshape(T, H)
    wgp = jnp.zeros((H, 128), jnp.float32).at[:, :E].set(Wg.T)
    topi_p, topw_p = _router(xf, wgp)
    topi = topi_p[:, :K]
    topw = topw_p[:, :K]

    flat_e = topi.reshape(-1)
    oh = (flat_e[:, None] == jnp.arange(E)[None, :]).astype(jnp.int32)
    rank = jnp.take_along_axis(jnp.cumsum(oh, axis=0), flat_e[:, None], 1)[:, 0] - 1
    counts = oh.sum(axis=0)
    padded = ((counts + M - 1) // M) * M
    pstart = jnp.concatenate([jnp.zeros(1, padded.dtype), jnp.cumsum(padded)])[:E]
    dest = (pstart[flat_e] + rank).astype(jnp.int32)
    src_tok = jnp.zeros(PN, jnp.int32).at[dest].set(jnp.arange(N, dtype=jnp.int32) // K)
    wslot3 = jnp.zeros(PN, jnp.float32).at[dest].set(topw.reshape(-1)).reshape(NT, 1, M)
    tile_expert = (jnp.sum(jnp.arange(NT)[:, None] * M >= pstart[None, :], axis=1)
                   - 1).astype(jnp.int32)
    cidx = jnp.concatenate([dest[0::K], dest[1::K]])

    xg = _dispatch(xf, src_tok)
    shared = _shared(xf, Ws1, Ws3, Ws2)
    yg = _ffn(tile_expert, xg, wslot3, W1, W3, W2)
    yt2 = _combine_gather(yg, cidx).reshape(2, T, H)
    out = _final_add(shared, yt2)
    return out.reshape(B, S, H)


# speedup vs baseline: 1.3021x; 1.2442x over previous
import functools

import jax
import jax.numpy as jnp
from jax import lax
from jax.experimental import pallas as pl
from jax.experimental.pallas import tpu as pltpu
from jax.experimental.pallas import tpu_sc as plsc

B, S, H = 2, 2048, 1024
E, K = 16, 2
FE, FS = 512, 1408
T = B * S
N = T * K
M = 256
NT = N // M + E
PN = NT * M

NC, NS = 2, 16
NW = NC * NS


def _router_body(x_ref, wg_ref, oi_ref, ow_ref):
    logits = lax.dot_general(x_ref[...], wg_ref[...], (((1,), (0,)), ((), ())),
                             preferred_element_type=jnp.float32)
    lane = lax.broadcasted_iota(jnp.int32, logits.shape, 1)
    logits = jnp.where(lane < E, logits, -1e30)
    m1 = jnp.max(logits, axis=1, keepdims=True)
    i1 = jnp.min(jnp.where(logits == m1, lane, 127), axis=1, keepdims=True)
    l2 = jnp.where(lane == i1, -1e30, logits)
    m2 = jnp.max(l2, axis=1, keepdims=True)
    i2 = jnp.min(jnp.where(l2 == m2, lane, 127), axis=1, keepdims=True)
    e2 = jnp.exp(m2 - m1)
    wa = 1.0 / (1.0 + e2)
    wb = 1.0 - wa
    oi_ref[...] = jnp.where(lane == 0, i1, jnp.where(lane == 1, i2, 0))
    ow_ref[...] = jnp.where(lane == 0, wa, jnp.where(lane == 1, wb, 0.0))


def _router(xf, wgp):
    bt = 512
    return pl.pallas_call(
        _router_body,
        grid=(T // bt,),
        in_specs=[
            pl.BlockSpec((bt, H), lambda i: (i, 0)),
            pl.BlockSpec((H, 128), lambda i: (0, 0)),
        ],
        out_specs=[
            pl.BlockSpec((bt, 128), lambda i: (i, 0)),
            pl.BlockSpec((bt, 128), lambda i: (i, 0)),
        ],
        out_shape=[
            jax.ShapeDtypeStruct((T, 128), jnp.int32),
            jax.ShapeDtypeStruct((T, 128), jnp.float32),
        ],
    )(xf, wgp)


_SC_MESH = plsc.VectorSubcoreMesh(core_axis_name="c", subcore_axis_name="s",
                                  num_cores=NC, num_subcores=NS)


def _make_gather(n_out, table_rows, ch):
    per_w = n_out // NW
    nch = per_w // ch
    assert per_w % ch == 0 and n_out % NW == 0

    @functools.partial(
        pl.kernel, mesh=_SC_MESH,
        out_type=jax.ShapeDtypeStruct((n_out, H), jnp.float32),
        scratch_types=[
            pltpu.VMEM((per_w,), jnp.int32),
            pltpu.VMEM((ch, H), jnp.float32),
            pltpu.VMEM((ch, H), jnp.float32),
            pltpu.SemaphoreType.DMA,
            pltpu.SemaphoreType.DMA,
            pltpu.SemaphoreType.DMA,
            pltpu.SemaphoreType.DMA,
        ],
    )
    def gather_kernel(table_hbm, idx_hbm, out_hbm, idx_v, rows0, rows1,
                      gsem0, gsem1, ssem0, ssem1):
        wid = lax.axis_index("s") * NC + lax.axis_index("c")
        base = wid * per_w
        pltpu.sync_copy(idx_hbm.at[pl.ds(base, per_w)], idx_v)
        bufs = (rows0, rows1)
        gsems = (gsem0, gsem1)
        ssems = (ssem0, ssem1)

        def gather(c, b):
            return pltpu.async_copy(
                table_hbm.at[idx_v.at[pl.ds(c * ch, ch)]], bufs[b], gsems[b])

        stores = [None, None]
        gs = [gather(0, 0), None]
        for c in range(nch):
            b = c & 1
            nb = 1 - b
            if c + 1 < nch:
                if stores[nb] is not None:
                    stores[nb].wait()
                gs[nb] = gather(c + 1, nb)
            gs[b].wait()
            stores[b] = pltpu.async_copy(
                bufs[b], out_hbm.at[pl.ds(base + c * ch, ch)], ssems[b])
        for st in stores:
            if st is not None:
                st.wait()

    return gather_kernel


_dispatch = _make_gather(PN, T, 48)
_combine_gather = _make_gather(N, PN, 32)


def _ffn_body(te_ref, x_ref, w_ref, w1_ref, w3_ref, w2_ref, o_ref):
    del te_ref
    x = x_ref[...].astype(jnp.bfloat16)
    h1 = lax.dot_general(x, w1_ref[0], (((1,), (0,)), ((), ())),
                         preferred_element_type=jnp.float32)
    h3 = lax.dot_general(x, w3_ref[0], (((1,), (0,)), ((), ())),
                         preferred_element_type=jnp.float32)
    w = w_ref[0, 0, :][:, None]
    hh = h1 * lax.logistic(h1) * h3 * w
    o_ref[...] = lax.dot_general(hh.astype(jnp.bfloat16), w2_ref[0],
                                 (((1,), (0,)), ((), ())),
                                 preferred_element_type=jnp.float32)


def _ffn(tile_expert, xg, wslot3, w1t, w3t, w2t):
    grid_spec = pltpu.PrefetchScalarGridSpec(
        num_scalar_prefetch=1,
        grid=(NT,),
        in_specs=[
            pl.BlockSpec((M, H), lambda i, te: (i, 0)),
            pl.BlockSpec((1, 1, M), lambda i, te: (i, 0, 0)),
            pl.BlockSpec((1, H, FE), lambda i, te: (te[i], 0, 0)),
            pl.BlockSpec((1, H, FE), lambda i, te: (te[i], 0, 0)),
            pl.BlockSpec((1, FE, H), lambda i, te: (te[i], 0, 0)),
        ],
        out_specs=pl.BlockSpec((M, H), lambda i, te: (i, 0)),
    )
    return pl.pallas_call(
        _ffn_body,
        grid_spec=grid_spec,
        out_shape=jax.ShapeDtypeStruct((PN, H), jnp.float32),
    )(tile_expert, xg, wslot3, w1t, w3t, w2t)


def _shared_body(x_ref, w1_ref, w3_ref, w2_ref, o_ref):
    x = x_ref[...].astype(jnp.bfloat16)
    h1 = lax.dot_general(x, w1_ref[...], (((1,), (0,)), ((), ())),
                         preferred_element_type=jnp.float32)
    h3 = lax.dot_general(x, w3_ref[...], (((1,), (0,)), ((), ())),
                         preferred_element_type=jnp.float32)
    hh = h1 * lax.logistic(h1) * h3
    o_ref[...] = lax.dot_general(hh.astype(jnp.bfloat16), w2_ref[...],
                                 (((1,), (0,)), ((), ())),
                                 preferred_element_type=jnp.float32)


def _shared(xf, ws1t, ws3t, ws2t):
    bt = 512
    return pl.pallas_call(
        _shared_body,
        grid=(T // bt,),
        in_specs=[
            pl.BlockSpec((bt, H), lambda i: (i, 0)),
            pl.BlockSpec((H, FS), lambda i: (0, 0)),
            pl.BlockSpec((H, FS), lambda i: (0, 0)),
            pl.BlockSpec((FS, H), lambda i: (0, 0)),
        ],
        out_specs=pl.BlockSpec((bt, H), lambda i: (i, 0)),
        out_shape=jax.ShapeDtypeStruct((T, H), jnp.float32),
    )(xf, ws1t, ws3t, ws2t)


def _add_body(s_ref, y0_ref, y1_ref, o_ref):
    o_ref[...] = s_ref[...] + y0_ref[0] + y1_ref[0]


def _final_add(shared, yt2):
    bt = 512
    return pl.pallas_call(
        _add_body,
        grid=(T // bt,),
        in_specs=[
            pl.BlockSpec((bt, H), lambda i: (i, 0)),
            pl.BlockSpec((1, bt, H), lambda i: (0, i, 0)),
            pl.BlockSpec((1, bt, H), lambda i: (1, i, 0)),
        ],
        out_specs=pl.BlockSpec((bt, H), lambda i: (i, 0)),
        out_shape=jax.ShapeDtypeStruct((T, H), jnp.float32),
    )(shared, yt2, yt2)


def kernel(x, Wg, W1, W2, W3, Ws1, Ws2, Ws3):
    xf = x.reshape(T, H)
    wgp = jnp.zeros((H, 128), jnp.float32).at[:, :E].set(Wg.T)
    topi_p, topw_p = _router(xf, wgp)
    topi = topi_p[:, :K]
    topw = topw_p[:, :K]

    flat_e = topi.reshape(-1)
    oh = (flat_e[:, None] == jnp.arange(E)[None, :]).astype(jnp.int32)
    rank = jnp.take_along_axis(jnp.cumsum(oh, axis=0), flat_e[:, None], 1)[:, 0] - 1
    counts = oh.sum(axis=0)
    padded = ((counts + M - 1) // M) * M
    pstart = jnp.concatenate([jnp.zeros(1, padded.dtype), jnp.cumsum(padded)])[:E]
    dest = (pstart[flat_e] + rank).astype(jnp.int32)
    pad_fill = jnp.arange(PN, dtype=jnp.int32) % T
    src_tok = pad_fill.at[dest].set(jnp.arange(N, dtype=jnp.int32) // K)
    wslot3 = jnp.zeros(PN, jnp.float32).at[dest].set(topw.reshape(-1)).reshape(NT, 1, M)
    tile_expert = (jnp.sum(jnp.arange(NT)[:, None] * M >= pstart[None, :], axis=1)
                   - 1).astype(jnp.int32)
    cidx = jnp.concatenate([dest[0::K], dest[1::K]])

    w1t = jnp.swapaxes(W1, 1, 2).astype(jnp.bfloat16)
    w3t = jnp.swapaxes(W3, 1, 2).astype(jnp.bfloat16)
    w2t = jnp.swapaxes(W2, 1, 2).astype(jnp.bfloat16)
    ws1t = Ws1.T.astype(jnp.bfloat16)
    ws3t = Ws3.T.astype(jnp.bfloat16)
    ws2t = Ws2.T.astype(jnp.bfloat16)

    shared = _shared(xf, ws1t, ws3t, ws2t)
    xg = _dispatch(xf, src_tok)
    yg = _ffn(tile_expert, xg, wslot3, w1t, w3t, w2t)
    yt2 = _combine_gather(yg, cidx).reshape(2, T, H)
    out = _final_add(shared, yt2)
    return out.reshape(B, S, H)


# speedup vs baseline: 1.7046x; 1.3092x over previous
import functools

import jax
import jax.numpy as jnp
from jax import lax
from jax.experimental import pallas as pl
from jax.experimental.pallas import tpu as pltpu
from jax.experimental.pallas import tpu_sc as plsc

B, S, H = 2, 2048, 1024
E, K = 16, 2
FE, FS = 512, 1408
T = B * S
N = T * K
M = 256
NT = N // M + E
PN = NT * M

NC, NS = 2, 16
NW = NC * NS


def _router_body(x_ref, wg_ref, oi_ref, ow_ref):
    logits = lax.dot_general(x_ref[...], wg_ref[...], (((1,), (0,)), ((), ())),
                             preferred_element_type=jnp.float32)
    lane = lax.broadcasted_iota(jnp.int32, logits.shape, 1)
    logits = jnp.where(lane < E, logits, -1e30)
    m1 = jnp.max(logits, axis=1, keepdims=True)
    i1 = jnp.min(jnp.where(logits == m1, lane, 127), axis=1, keepdims=True)
    l2 = jnp.where(lane == i1, -1e30, logits)
    m2 = jnp.max(l2, axis=1, keepdims=True)
    i2 = jnp.min(jnp.where(l2 == m2, lane, 127), axis=1, keepdims=True)
    e2 = jnp.exp(m2 - m1)
    wa = 1.0 / (1.0 + e2)
    wb = 1.0 - wa
    oi_ref[...] = jnp.where(lane == 0, i1, jnp.where(lane == 1, i2, 0))
    ow_ref[...] = jnp.where(lane == 0, wa, jnp.where(lane == 1, wb, 0.0))


def _router(xf, wgp):
    bt = 512
    return pl.pallas_call(
        _router_body,
        grid=(T // bt,),
        in_specs=[
            pl.BlockSpec((bt, H), lambda i: (i, 0)),
            pl.BlockSpec((H, 128), lambda i: (0, 0)),
        ],
        out_specs=[
            pl.BlockSpec((bt, 128), lambda i: (i, 0)),
            pl.BlockSpec((bt, 128), lambda i: (i, 0)),
        ],
        out_shape=[
            jax.ShapeDtypeStruct((T, 128), jnp.int32),
            jax.ShapeDtypeStruct((T, 128), jnp.float32),
        ],
    )(xf, wgp)


_SC_MESH = plsc.VectorSubcoreMesh(core_axis_name="c", subcore_axis_name="s",
                                  num_cores=NC, num_subcores=NS)
_D_PW = T // NW
_D_CH = 32
_D_NCH = _D_PW // _D_CH


@functools.partial(
    pl.kernel, mesh=_SC_MESH,
    out_type=jax.ShapeDtypeStruct((PN, H), jnp.float32),
    scratch_types=[
        pltpu.VMEM((_D_NCH, _D_CH), jnp.int32),
        pltpu.VMEM((_D_NCH, _D_CH), jnp.int32),
        pltpu.VMEM((_D_CH, H), jnp.float32),
        pltpu.VMEM((_D_CH, H), jnp.float32),
        pltpu.SemaphoreType.DMA,
        pltpu.SemaphoreType.DMA,
        pltpu.SemaphoreType.DMA,
        pltpu.SemaphoreType.DMA,
    ],
)
def _dispatch(xfb_hbm, de_hbm, do_hbm, xg_hbm, ie_v, io_v, rows0, rows1,
              lsem0, lsem1, ssem0, ssem1):
    wid = lax.axis_index("s") * NC + lax.axis_index("c")
    base = wid * _D_PW
    pltpu.sync_copy(de_hbm.at[wid], ie_v)
    pltpu.sync_copy(do_hbm.at[wid], io_v)
    bufs = (rows0, rows1)
    lsems = (lsem0, lsem1)
    ssems = (ssem0, ssem1)

    def load(c, b):
        return pltpu.async_copy(
            xfb_hbm.at[pl.ds(base + c * _D_CH, _D_CH)], bufs[b], lsems[b])

    stores = [None, None]
    ls = [load(0, 0), None]
    for c in range(_D_NCH):
        b = c & 1
        nb = 1 - b
        if c + 1 < _D_NCH:
            if stores[nb] is not None:
                stores[nb][0].wait()
                stores[nb][1].wait()
            ls[nb] = load(c + 1, nb)
        ls[b].wait()
        se = pltpu.async_copy(bufs[b], xg_hbm.at[ie_v.at[c]], ssems[b])
        so = pltpu.async_copy(bufs[b], xg_hbm.at[io_v.at[c]], ssems[b])
        stores[b] = (se, so)
    for st in stores:
        if st is not None:
            st[0].wait()
            st[1].wait()


_G_PW = N // NW
_G_CH = 32
_G_NCH = _G_PW // _G_CH


@functools.partial(
    pl.kernel, mesh=_SC_MESH,
    out_type=jax.ShapeDtypeStruct((N, H), jnp.float32),
    scratch_types=[
        pltpu.VMEM((_G_PW,), jnp.int32),
        pltpu.VMEM((_G_CH, H), jnp.float32),
        pltpu.VMEM((_G_CH, H), jnp.float32),
        pltpu.SemaphoreType.DMA,
        pltpu.SemaphoreType.DMA,
        pltpu.SemaphoreType.DMA,
        pltpu.SemaphoreType.DMA,
    ],
)
def _combine_gather(yg_hbm, idx_hbm, out_hbm, idx_v, rows0, rows1,
                    gsem0, gsem1, ssem0, ssem1):
    wid = lax.axis_index("s") * NC + lax.axis_index("c")
    base = wid * _G_PW
    pltpu.sync_copy(idx_hbm.at[pl.ds(base, _G_PW)], idx_v)
    bufs = (rows0, rows1)
    gsems = (gsem0, gsem1)
    ssems = (ssem0, ssem1)

    def gather(c, b):
        return pltpu.async_copy(
            yg_hbm.at[idx_v.at[pl.ds(c * _G_CH, _G_CH)]], bufs[b], gsems[b])

    stores = [None, None]
    gs = [gather(0, 0), None]
    for c in range(_G_NCH):
        b = c & 1
        nb = 1 - b
        if c + 1 < _G_NCH:
            if stores[nb] is not None:
                stores[nb].wait()
            gs[nb] = gather(c + 1, nb)
        gs[b].wait()
        stores[b] = pltpu.async_copy(
            bufs[b], out_hbm.at[pl.ds(base + c * _G_CH, _G_CH)], ssems[b])
    for st in stores:
        if st is not None:
            st.wait()


def _ffn_body(te_ref, x_ref, w1_ref, w3_ref, w2_ref, o_ref):
    del te_ref
    x = x_ref[...].astype(jnp.bfloat16)
    h1 = lax.dot_general(x, w1_ref[0], (((1,), (1,)), ((), ())),
                         preferred_element_type=jnp.float32)
    h3 = lax.dot_general(x, w3_ref[0], (((1,), (1,)), ((), ())),
                         preferred_element_type=jnp.float32)
    hh = h1 * lax.logistic(h1) * h3
    o_ref[...] = lax.dot_general(hh.astype(jnp.bfloat16), w2_ref[0],
                                 (((1,), (1,)), ((), ())),
                                 preferred_element_type=jnp.float32)


def _ffn(tile_expert, xg, w1b, w3b, w2b):
    grid_spec = pltpu.PrefetchScalarGridSpec(
        num_scalar_prefetch=1,
        grid=(NT,),
        in_specs=[
            pl.BlockSpec((M, H), lambda i, te: (i, 0)),
            pl.BlockSpec((1, FE, H), lambda i, te: (te[i], 0, 0)),
            pl.BlockSpec((1, FE, H), lambda i, te: (te[i], 0, 0)),
            pl.BlockSpec((1, H, FE), lambda i, te: (te[i], 0, 0)),
        ],
        out_specs=pl.BlockSpec((M, H), lambda i, te: (i, 0)),
    )
    return pl.pallas_call(
        _ffn_body,
        grid_spec=grid_spec,
        out_shape=jax.ShapeDtypeStruct((PN, H), jnp.float32),
    )(tile_expert, xg, w1b, w3b, w2b)


def _shared_body(x_ref, w1_ref, w3_ref, w2_ref, o_ref):
    x = x_ref[...].astype(jnp.bfloat16)
    h1 = lax.dot_general(x, w1_ref[...], (((1,), (1,)), ((), ())),
                         preferred_element_type=jnp.float32)
    h3 = lax.dot_general(x, w3_ref[...], (((1,), (1,)), ((), ())),
                         preferred_element_type=jnp.float32)
    hh = h1 * lax.logistic(h1) * h3
    o_ref[...] = lax.dot_general(hh.astype(jnp.bfloat16), w2_ref[...],
                                 (((1,), (1,)), ((), ())),
                                 preferred_element_type=jnp.float32)


def _shared(xf, ws1b, ws3b, ws2b):
    bt = 512
    return pl.pallas_call(
        _shared_body,
        grid=(T // bt,),
        in_specs=[
            pl.BlockSpec((bt, H), lambda i: (i, 0)),
            pl.BlockSpec((FS, H), lambda i: (0, 0)),
            pl.BlockSpec((FS, H), lambda i: (0, 0)),
            pl.BlockSpec((H, FS), lambda i: (0, 0)),
        ],
        out_specs=pl.BlockSpec((bt, H), lambda i: (i, 0)),
        out_shape=jax.ShapeDtypeStruct((T, H), jnp.float32),
    )(xf, ws1b, ws3b, ws2b)


def _add_body(s_ref, w_ref, y0_ref, y1_ref, o_ref):
    w0 = w_ref[:, 0:1]
    w1 = w_ref[:, 1:2]
    o_ref[...] = s_ref[...] + w0 * y0_ref[0] + w1 * y1_ref[0]


def _final_add(shared, topw_p, yt2):
    bt = 512
    return pl.pallas_call(
        _add_body,
        grid=(T // bt,),
        in_specs=[
            pl.BlockSpec((bt, H), lambda i: (i, 0)),
            pl.BlockSpec((bt, 128), lambda i: (i, 0)),
            pl.BlockSpec((1, bt, H), lambda i: (0, i, 0)),
            pl.BlockSpec((1, bt, H), lambda i: (1, i, 0)),
        ],
        out_specs=pl.BlockSpec((bt, H), lambda i: (i, 0)),
        out_shape=jax.ShapeDtypeStruct((T, H), jnp.float32),
    )(shared, topw_p, yt2, yt2)


def kernel(x, Wg, W1, W2, W3, Ws1, Ws2, Ws3):
    xf = x.reshape(T, H)
    wgp = jnp.zeros((H, 128), jnp.float32).at[:, :E].set(Wg.T)
    topi_p, topw_p = _router(xf, wgp)
    topi = topi_p[:, :K]

    flat_e = topi.reshape(-1)
    oh = (flat_e[:, None] == jnp.arange(E)[None, :]).astype(jnp.int32)
    rank = jnp.take_along_axis(jnp.cumsum(oh, axis=0), flat_e[:, None], 1)[:, 0] - 1
    counts = oh.sum(axis=0)
    padded = ((counts + M - 1) // M) * M
    pstart = jnp.concatenate([jnp.zeros(1, padded.dtype), jnp.cumsum(padded)])[:E]
    dest = (pstart[flat_e] + rank).astype(jnp.int32)
    tile_expert = (jnp.sum(jnp.arange(NT)[:, None] * M >= pstart[None, :], axis=1)
                   - 1).astype(jnp.int32)
    de3 = dest[0::K].reshape(NW, _D_NCH, _D_CH)
    do3 = dest[1::K].reshape(NW, _D_NCH, _D_CH)
    cidx = jnp.concatenate([dest[0::K], dest[1::K]])

    w1b = W1.astype(jnp.bfloat16)
    w3b = W3.astype(jnp.bfloat16)
    w2b = W2.astype(jnp.bfloat16)
    ws1b = Ws1.astype(jnp.bfloat16)
    ws3b = Ws3.astype(jnp.bfloat16)
    ws2b = Ws2.astype(jnp.bfloat16)

    xg = _dispatch(xf, de3, do3)
    yg = _ffn(tile_expert, xg, w1b, w3b, w2b)
    shared = _shared(xf, ws1b, ws3b, ws2b)
    yt2 = _combine_gather(yg, cidx).reshape(2, T, H)
    out = _final_add(shared, topw_p, yt2)
    return out.reshape(B, S, H)


# speedup vs baseline: 2.0417x; 1.1978x over previous
import functools

import jax
import jax.numpy as jnp
from jax import lax
from jax.experimental import pallas as pl
from jax.experimental.pallas import tpu as pltpu
from jax.experimental.pallas import tpu_sc as plsc

B, S, H = 2, 2048, 1024
E, K = 16, 2
FE, FS = 512, 1408
T = B * S
N = T * K
M = 256
NT = N // M + E
PN = NT * M

NC, NS = 2, 16
NW = NC * NS


def _router_body(x_ref, wg_ref, oi_ref, ow_ref):
    logits = lax.dot_general(x_ref[...], wg_ref[...], (((1,), (0,)), ((), ())),
                             preferred_element_type=jnp.float32)
    lane = lax.broadcasted_iota(jnp.int32, logits.shape, 1)
    logits = jnp.where(lane < E, logits, -1e30)
    m1 = jnp.max(logits, axis=1, keepdims=True)
    i1 = jnp.min(jnp.where(logits == m1, lane, 127), axis=1, keepdims=True)
    l2 = jnp.where(lane == i1, -1e30, logits)
    m2 = jnp.max(l2, axis=1, keepdims=True)
    i2 = jnp.min(jnp.where(l2 == m2, lane, 127), axis=1, keepdims=True)
    e2 = jnp.exp(m2 - m1)
    wa = 1.0 / (1.0 + e2)
    wb = 1.0 - wa
    oi_ref[...] = jnp.where(lane == 0, i1, jnp.where(lane == 1, i2, 0))
    ow_ref[...] = jnp.where(lane == 0, wa, jnp.where(lane == 1, wb, 0.0))


def _router(xf, wgp):
    bt = 512
    return pl.pallas_call(
        _router_body,
        grid=(T // bt,),
        in_specs=[
            pl.BlockSpec((bt, H), lambda i: (i, 0)),
            pl.BlockSpec((H, 128), lambda i: (0, 0)),
        ],
        out_specs=[
            pl.BlockSpec((bt, 128), lambda i: (i, 0)),
            pl.BlockSpec((bt, 128), lambda i: (i, 0)),
        ],
        out_shape=[
            jax.ShapeDtypeStruct((T, 128), jnp.int32),
            jax.ShapeDtypeStruct((T, 128), jnp.float32),
        ],
    )(xf, wgp)


_SC_MESH = plsc.VectorSubcoreMesh(core_axis_name="c", subcore_axis_name="s",
                                  num_cores=NC, num_subcores=NS)
_D_PW = T // NW
_D_CH = 32
_D_NCH = _D_PW // _D_CH


@functools.partial(
    pl.kernel, mesh=_SC_MESH,
    out_type=jax.ShapeDtypeStruct((PN, H), jnp.float32),
    scratch_types=[
        pltpu.VMEM((_D_NCH, _D_CH), jnp.int32),
        pltpu.VMEM((_D_NCH, _D_CH), jnp.int32),
        pltpu.VMEM((_D_CH, H), jnp.float32),
        pltpu.VMEM((_D_CH, H), jnp.float32),
        pltpu.SemaphoreType.DMA,
        pltpu.SemaphoreType.DMA,
        pltpu.SemaphoreType.DMA,
        pltpu.SemaphoreType.DMA,
    ],
)
def _dispatch(xfb_hbm, de_hbm, do_hbm, xg_hbm, ie_v, io_v, rows0, rows1,
              lsem0, lsem1, ssem0, ssem1):
    wid = lax.axis_index("s") * NC + lax.axis_index("c")
    base = wid * _D_PW
    pltpu.sync_copy(de_hbm.at[wid], ie_v)
    pltpu.sync_copy(do_hbm.at[wid], io_v)
    bufs = (rows0, rows1)
    lsems = (lsem0, lsem1)
    ssems = (ssem0, ssem1)

    def load(c, b):
        return pltpu.async_copy(
            xfb_hbm.at[pl.ds(base + c * _D_CH, _D_CH)], bufs[b], lsems[b])

    stores = [None, None]
    ls = [load(0, 0), None]
    for c in range(_D_NCH):
        b = c & 1
        nb = 1 - b
        if c + 1 < _D_NCH:
            if stores[nb] is not None:
                stores[nb][0].wait()
                stores[nb][1].wait()
            ls[nb] = load(c + 1, nb)
        ls[b].wait()
        se = pltpu.async_copy(bufs[b], xg_hbm.at[ie_v.at[c]], ssems[b])
        so = pltpu.async_copy(bufs[b], xg_hbm.at[io_v.at[c]], ssems[b])
        stores[b] = (se, so)
    for st in stores:
        if st is not None:
            st[0].wait()
            st[1].wait()


_G_PW = N // NW
_G_CH = 32
_G_NCH = _G_PW // _G_CH


@functools.partial(
    pl.kernel, mesh=_SC_MESH,
    out_type=jax.ShapeDtypeStruct((N, H), jnp.float32),
    scratch_types=[
        pltpu.VMEM((_G_PW,), jnp.int32),
        pltpu.VMEM((_G_CH, H), jnp.float32),
        pltpu.VMEM((_G_CH, H), jnp.float32),
        pltpu.SemaphoreType.DMA,
        pltpu.SemaphoreType.DMA,
        pltpu.SemaphoreType.DMA,
        pltpu.SemaphoreType.DMA,
    ],
)
def _combine_gather(yg_hbm, idx_hbm, out_hbm, idx_v, rows0, rows1,
                    gsem0, gsem1, ssem0, ssem1):
    wid = lax.axis_index("s") * NC + lax.axis_index("c")
    base = wid * _G_PW
    pltpu.sync_copy(idx_hbm.at[pl.ds(base, _G_PW)], idx_v)
    bufs = (rows0, rows1)
    gsems = (gsem0, gsem1)
    ssems = (ssem0, ssem1)

    def gather(c, b):
        return pltpu.async_copy(
            yg_hbm.at[idx_v.at[pl.ds(c * _G_CH, _G_CH)]], bufs[b], gsems[b])

    stores = [None, None]
    gs = [gather(0, 0), None]
    for c in range(_G_NCH):
        b = c & 1
        nb = 1 - b
        if c + 1 < _G_NCH:
            if stores[nb] is not None:
                stores[nb].wait()
            gs[nb] = gather(c + 1, nb)
        gs[b].wait()
        stores[b] = pltpu.async_copy(
            bufs[b], out_hbm.at[pl.ds(base + c * _G_CH, _G_CH)], ssems[b])
    for st in stores:
        if st is not None:
            st.wait()


def _ffn_body(te_ref, x_ref, w1_ref, w3_ref, w2_ref, o_ref):
    del te_ref
    x = x_ref[...]
    h1 = lax.dot_general(x, w1_ref[0], (((1,), (1,)), ((), ())),
                         preferred_element_type=jnp.float32)
    h3 = lax.dot_general(x, w3_ref[0], (((1,), (1,)), ((), ())),
                         preferred_element_type=jnp.float32)
    hh = h1 * lax.logistic(h1) * h3
    o_ref[...] = lax.dot_general(hh, w2_ref[0], (((1,), (1,)), ((), ())),
                                 preferred_element_type=jnp.float32)


def _ffn(tile_expert, xg, w1b, w3b, w2b):
    grid_spec = pltpu.PrefetchScalarGridSpec(
        num_scalar_prefetch=1,
        grid=(NT,),
        in_specs=[
            pl.BlockSpec((M, H), lambda i, te: (i, 0)),
            pl.BlockSpec((1, FE, H), lambda i, te: (te[i], 0, 0)),
            pl.BlockSpec((1, FE, H), lambda i, te: (te[i], 0, 0)),
            pl.BlockSpec((1, H, FE), lambda i, te: (te[i], 0, 0)),
        ],
        out_specs=pl.BlockSpec((M, H), lambda i, te: (i, 0)),
    )
    return pl.pallas_call(
        _ffn_body,
        grid_spec=grid_spec,
        out_shape=jax.ShapeDtypeStruct((PN, H), jnp.float32),
    )(tile_expert, xg, w1b, w3b, w2b)


def _shared_body(x_ref, w1_ref, w3_ref, w2_ref, o_ref):
    x = x_ref[...]
    h1 = lax.dot_general(x, w1_ref[...], (((1,), (1,)), ((), ())),
                         preferred_element_type=jnp.float32)
    h3 = lax.dot_general(x, w3_ref[...], (((1,), (1,)), ((), ())),
                         preferred_element_type=jnp.float32)
    hh = h1 * lax.logistic(h1) * h3
    o_ref[...] = lax.dot_general(hh, w2_ref[...], (((1,), (1,)), ((), ())),
                                 preferred_element_type=jnp.float32)


def _shared(xf, ws1b, ws3b, ws2b):
    bt = 512
    return pl.pallas_call(
        _shared_body,
        grid=(T // bt,),
        in_specs=[
            pl.BlockSpec((bt, H), lambda i: (i, 0)),
            pl.BlockSpec((FS, H), lambda i: (0, 0)),
            pl.BlockSpec((FS, H), lambda i: (0, 0)),
            pl.BlockSpec((H, FS), lambda i: (0, 0)),
        ],
        out_specs=pl.BlockSpec((bt, H), lambda i: (i, 0)),
        out_shape=jax.ShapeDtypeStruct((T, H), jnp.float32),
    )(xf, ws1b, ws3b, ws2b)


def _add_body(s_ref, w_ref, y0_ref, y1_ref, o_ref):
    w0 = w_ref[:, 0:1]
    w1 = w_ref[:, 1:2]
    o_ref[...] = s_ref[...] + w0 * y0_ref[0] + w1 * y1_ref[0]


def _final_add(shared, topw_p, yt2):
    bt = 512
    return pl.pallas_call(
        _add_body,
        grid=(T // bt,),
        in_specs=[
            pl.BlockSpec((bt, H), lambda i: (i, 0)),
            pl.BlockSpec((bt, 128), lambda i: (i, 0)),
            pl.BlockSpec((1, bt, H), lambda i: (0, i, 0)),
            pl.BlockSpec((1, bt, H), lambda i: (1, i, 0)),
        ],
        out_specs=pl.BlockSpec((bt, H), lambda i: (i, 0)),
        out_shape=jax.ShapeDtypeStruct((T, H), jnp.float32),
    )(shared, topw_p, yt2, yt2)


def kernel(x, Wg, W1, W2, W3, Ws1, Ws2, Ws3):
    xf = x.reshape(T, H)
    wgp = jnp.zeros((H, 128), jnp.float32).at[:, :E].set(Wg.T)
    topi_p, topw_p = _router(xf, wgp)
    topi = topi_p[:, :K]

    flat_e = topi.reshape(-1)
    oh = (flat_e[:, None] == jnp.arange(E)[None, :]).astype(jnp.int32)
    rank = jnp.take_along_axis(jnp.cumsum(oh, axis=0), flat_e[:, None], 1)[:, 0] - 1
    counts = oh.sum(axis=0)
    padded = ((counts + M - 1) // M) * M
    pstart = jnp.concatenate([jnp.zeros(1, padded.dtype), jnp.cumsum(padded)])[:E]
    dest = (pstart[flat_e] + rank).astype(jnp.int32)
    tile_expert = (jnp.sum(jnp.arange(NT)[:, None] * M >= pstart[None, :], axis=1)
                   - 1).astype(jnp.int32)
    de3 = dest[0::K].reshape(NW, _D_NCH, _D_CH)
    do3 = dest[1::K].reshape(NW, _D_NCH, _D_CH)
    cidx = jnp.concatenate([dest[0::K], dest[1::K]])

    xg = _dispatch(xf, de3, do3)
    yg = _ffn(tile_expert, xg, W1, W3, W2)
    shared = _shared(xf, Ws1, Ws3, Ws2)
    yt2 = _combine_gather(yg, cidx).reshape(2, T, H)
    out = _final_add(shared, topw_p, yt2)
    return out.reshape(B, S, H)


# speedup vs baseline: 2.1825x; 1.0690x over previous
import functools

import jax
import jax.numpy as jnp
from jax import lax
from jax.experimental import pallas as pl
from jax.experimental.pallas import tpu as pltpu
from jax.experimental.pallas import tpu_sc as plsc

B, S, H = 2, 2048, 1024
E, K = 16, 2
FE, FS = 512, 1408
T = B * S
N = T * K
M = 256
NT = N // M + E
PN = NT * M

NC, NS = 2, 16
NW = NC * NS


def _pack_rows(a):
    ab = a.astype(jnp.bfloat16).astype(jnp.float32)
    lo = lax.bitcast_convert_type(ab[:, : H // 2], jnp.uint32)
    hi = lax.bitcast_convert_type(ab[:, H // 2 :], jnp.uint32)
    return lax.bitcast_convert_type(hi | (lo >> 16), jnp.float32)


def _unpack_rows(p):
    u = lax.bitcast_convert_type(p, jnp.uint32)
    lo = lax.bitcast_convert_type(u << 16, jnp.float32)
    hi = lax.bitcast_convert_type(u & jnp.uint32(0xFFFF0000), jnp.float32)
    return jnp.concatenate([lo, hi], axis=1)


def _router_body(x_ref, wg_ref, oi_ref, ow_ref, ox_ref):
    logits = lax.dot_general(x_ref[...], wg_ref[...], (((1,), (0,)), ((), ())),
                             preferred_element_type=jnp.float32)
    lane = lax.broadcasted_iota(jnp.int32, logits.shape, 1)
    logits = jnp.where(lane < E, logits, -1e30)
    m1 = jnp.max(logits, axis=1, keepdims=True)
    i1 = jnp.min(jnp.where(logits == m1, lane, 127), axis=1, keepdims=True)
    l2 = jnp.where(lane == i1, -1e30, logits)
    m2 = jnp.max(l2, axis=1, keepdims=True)
    i2 = jnp.min(jnp.where(l2 == m2, lane, 127), axis=1, keepdims=True)
    e2 = jnp.exp(m2 - m1)
    wa = 1.0 / (1.0 + e2)
    wb = 1.0 - wa
    oi_ref[...] = jnp.where(lane == 0, i1, jnp.where(lane == 1, i2, 0))
    ow_ref[...] = jnp.where(lane == 0, wa, jnp.where(lane == 1, wb, 0.0))
    ox_ref[...] = _pack_rows(x_ref[...])


def _router(xf, wgp):
    bt = 512
    return pl.pallas_call(
        _router_body,
        grid=(T // bt,),
        in_specs=[
            pl.BlockSpec((bt, H), lambda i: (i, 0)),
            pl.BlockSpec((H, 128), lambda i: (0, 0)),
        ],
        out_specs=[
            pl.BlockSpec((bt, 128), lambda i: (i, 0)),
            pl.BlockSpec((bt, 128), lambda i: (i, 0)),
            pl.BlockSpec((bt, H // 2), lambda i: (i, 0)),
        ],
        out_shape=[
            jax.ShapeDtypeStruct((T, 128), jnp.int32),
            jax.ShapeDtypeStruct((T, 128), jnp.float32),
            jax.ShapeDtypeStruct((T, H // 2), jnp.float32),
        ],
    )(xf, wgp)


_SC_MESH = plsc.VectorSubcoreMesh(core_axis_name="c", subcore_axis_name="s",
                                  num_cores=NC, num_subcores=NS)
_D_PW = T // NW
_D_CH = 32
_D_NCH = _D_PW // _D_CH


@functools.partial(
    pl.kernel, mesh=_SC_MESH,
    out_type=jax.ShapeDtypeStruct((PN, H // 2), jnp.float32),
    scratch_types=[
        pltpu.VMEM((_D_NCH, _D_CH), jnp.int32),
        pltpu.VMEM((_D_NCH, _D_CH), jnp.int32),
        pltpu.VMEM((_D_CH, H // 2), jnp.float32),
        pltpu.VMEM((_D_CH, H // 2), jnp.float32),
        pltpu.SemaphoreType.DMA,
        pltpu.SemaphoreType.DMA,
        pltpu.SemaphoreType.DMA,
        pltpu.SemaphoreType.DMA,
    ],
)
def _dispatch(xfb_hbm, de_hbm, do_hbm, xg_hbm, ie_v, io_v, rows0, rows1,
              lsem0, lsem1, ssem0, ssem1):
    wid = lax.axis_index("s") * NC + lax.axis_index("c")
    base = wid * _D_PW
    pltpu.sync_copy(de_hbm.at[wid], ie_v)
    pltpu.sync_copy(do_hbm.at[wid], io_v)
    bufs = (rows0, rows1)
    lsems = (lsem0, lsem1)
    ssems = (ssem0, ssem1)

    def load(c, b):
        return pltpu.async_copy(
            xfb_hbm.at[pl.ds(base + c * _D_CH, _D_CH)], bufs[b], lsems[b])

    stores = [None, None]
    ls = [load(0, 0), None]
    for c in range(_D_NCH):
        b = c & 1
        nb = 1 - b
        if c + 1 < _D_NCH:
            if stores[nb] is not None:
                stores[nb][0].wait()
                stores[nb][1].wait()
            ls[nb] = load(c + 1, nb)
        ls[b].wait()
        se = pltpu.async_copy(bufs[b], xg_hbm.at[ie_v.at[c]], ssems[b])
        so = pltpu.async_copy(bufs[b], xg_hbm.at[io_v.at[c]], ssems[b])
        stores[b] = (se, so)
    for st in stores:
        if st is not None:
            st[0].wait()
            st[1].wait()


_G_PW = N // NW
_G_CH = 32
_G_NCH = _G_PW // _G_CH


@functools.partial(
    pl.kernel, mesh=_SC_MESH,
    out_type=jax.ShapeDtypeStruct((N, H // 2), jnp.float32),
    scratch_types=[
        pltpu.VMEM((_G_PW,), jnp.int32),
        pltpu.VMEM((_G_CH, H // 2), jnp.float32),
        pltpu.VMEM((_G_CH, H // 2), jnp.float32),
        pltpu.SemaphoreType.DMA,
        pltpu.SemaphoreType.DMA,
        pltpu.SemaphoreType.DMA,
        pltpu.SemaphoreType.DMA,
    ],
)
def _combine_gather(yg_hbm, idx_hbm, out_hbm, idx_v, rows0, rows1,
                    gsem0, gsem1, ssem0, ssem1):
    wid = lax.axis_index("s") * NC + lax.axis_index("c")
    base = wid * _G_PW
    pltpu.sync_copy(idx_hbm.at[pl.ds(base, _G_PW)], idx_v)
    bufs = (rows0, rows1)
    gsems = (gsem0, gsem1)
    ssems = (ssem0, ssem1)

    def gather(c, b):
        return pltpu.async_copy(
            yg_hbm.at[idx_v.at[pl.ds(c * _G_CH, _G_CH)]], bufs[b], gsems[b])

    stores = [None, None]
    gs = [gather(0, 0), None]
    for c in range(_G_NCH):
        b = c & 1
        nb = 1 - b
        if c + 1 < _G_NCH:
            if stores[nb] is not None:
                stores[nb].wait()
            gs[nb] = gather(c + 1, nb)
        gs[b].wait()
        stores[b] = pltpu.async_copy(
            bufs[b], out_hbm.at[pl.ds(base + c * _G_CH, _G_CH)], ssems[b])
    for st in stores:
        if st is not None:
            st.wait()


def _ffn_body(te_ref, x_ref, w1_ref, w3_ref, w2_ref, o_ref):
    del te_ref
    x = _unpack_rows(x_ref[...])
    h1 = lax.dot_general(x, w1_ref[0], (((1,), (1,)), ((), ())),
                         preferred_element_type=jnp.float32)
    h3 = lax.dot_general(x, w3_ref[0], (((1,), (1,)), ((), ())),
                         preferred_element_type=jnp.float32)
    hh = h1 * lax.logistic(h1) * h3
    y = lax.dot_general(hh, w2_ref[0], (((1,), (1,)), ((), ())),
                        preferred_element_type=jnp.float32)
    o_ref[...] = _pack_rows(y)


def _ffn(tile_expert, xg, w1b, w3b, w2b):
    grid_spec = pltpu.PrefetchScalarGridSpec(
        num_scalar_prefetch=1,
        grid=(NT,),
        in_specs=[
            pl.BlockSpec((M, H // 2), lambda i, te: (i, 0)),
            pl.BlockSpec((1, FE, H), lambda i, te: (te[i], 0, 0)),
            pl.BlockSpec((1, FE, H), lambda i, te: (te[i], 0, 0)),
            pl.BlockSpec((1, H, FE), lambda i, te: (te[i], 0, 0)),
        ],
        out_specs=pl.BlockSpec((M, H // 2), lambda i, te: (i, 0)),
    )
    return pl.pallas_call(
        _ffn_body,
        grid_spec=grid_spec,
        out_shape=jax.ShapeDtypeStruct((PN, H // 2), jnp.float32),
    )(tile_expert, xg, w1b, w3b, w2b)


def _shared_body(x_ref, w1_ref, w3_ref, w2_ref, o_ref):
    x = x_ref[...]
    h1 = lax.dot_general(x, w1_ref[...], (((1,), (1,)), ((), ())),
                         preferred_element_type=jnp.float32)
    h3 = lax.dot_general(x, w3_ref[...], (((1,), (1,)), ((), ())),
                         preferred_element_type=jnp.float32)
    hh = h1 * lax.logistic(h1) * h3
    o_ref[...] = lax.dot_general(hh, w2_ref[...], (((1,), (1,)), ((), ())),
                                 preferred_element_type=jnp.float32)


def _shared(xf, ws1b, ws3b, ws2b):
    bt = 512
    return pl.pallas_call(
        _shared_body,
        grid=(T // bt,),
        in_specs=[
            pl.BlockSpec((bt, H), lambda i: (i, 0)),
            pl.BlockSpec((FS, H), lambda i: (0, 0)),
            pl.BlockSpec((FS, H), lambda i: (0, 0)),
            pl.BlockSpec((H, FS), lambda i: (0, 0)),
        ],
        out_specs=pl.BlockSpec((bt, H), lambda i: (i, 0)),
        out_shape=jax.ShapeDtypeStruct((T, H), jnp.float32),
    )(xf, ws1b, ws3b, ws2b)


def _add_body(s_ref, w_ref, y0_ref, y1_ref, o_ref):
    w0 = w_ref[:, 0:1]
    w1 = w_ref[:, 1:2]
    y0 = _unpack_rows(y0_ref[0])
    y1 = _unpack_rows(y1_ref[0])
    o_ref[...] = s_ref[...] + w0 * y0 + w1 * y1


def _final_add(shared, topw_p, yt2):
    bt = 512
    return pl.pallas_call(
        _add_body,
        grid=(T // bt,),
        in_specs=[
            pl.BlockSpec((bt, H), lambda i: (i, 0)),
            pl.BlockSpec((bt, 128), lambda i: (i, 0)),
            pl.BlockSpec((1, bt, H // 2), lambda i: (0, i, 0)),
            pl.BlockSpec((1, bt, H // 2), lambda i: (1, i, 0)),
        ],
        out_specs=pl.BlockSpec((bt, H), lambda i: (i, 0)),
        out_shape=jax.ShapeDtypeStruct((T, H), jnp.float32),
    )(shared, topw_p, yt2, yt2)


def kernel(x, Wg, W1, W2, W3, Ws1, Ws2, Ws3):
    xf = x.reshape(T, H)
    wgp = jnp.zeros((H, 128), jnp.float32).at[:, :E].set(Wg.T)
    topi_p, topw_p, xfp = _router(xf, wgp)
    topi = topi_p[:, :K]

    flat_e = topi.reshape(-1)
    oh = (flat_e[:, None] == jnp.arange(E)[None, :]).astype(jnp.int32)
    rank = jnp.take_along_axis(jnp.cumsum(oh, axis=0), flat_e[:, None], 1)[:, 0] - 1
    counts = oh.sum(axis=0)
    padded = ((counts + M - 1) // M) * M
    pstart = jnp.concatenate([jnp.zeros(1, padded.dtype), jnp.cumsum(padded)])[:E]
    dest = (pstart[flat_e] + rank).astype(jnp.int32)
    tile_expert = (jnp.sum(jnp.arange(NT)[:, None] * M >= pstart[None, :], axis=1)
                   - 1).astype(jnp.int32)
    de3 = dest[0::K].reshape(NW, _D_NCH, _D_CH)
    do3 = dest[1::K].reshape(NW, _D_NCH, _D_CH)
    cidx = jnp.concatenate([dest[0::K], dest[1::K]])

    xg = _dispatch(xfp, de3, do3)
    yg = _ffn(tile_expert, xg, W1, W3, W2)
    shared = _shared(xf, Ws1, Ws3, Ws2)
    yt2 = _combine_gather(yg, cidx).reshape(2, T, H // 2)
    out = _final_add(shared, topw_p, yt2)
    return out.reshape(B, S, H)
